# baseline (speedup 1.0000x reference)
import functools

import jax
import jax.numpy as jnp
from jax import lax
from jax.experimental import pallas as pl
from jax.experimental.pallas import tpu as pltpu
from jax.experimental.pallas import tpu_sc as plsc

N = 10000
NP = 10240
E = 320000
NW = 32
EW = E // NW
BLK = 128
BLK2 = 256
NB = 80
NB2 = 40
EWP = NB * BLK
SLICE = NP // 16
F32 = jnp.float32


def _mesh():
    return plsc.VectorSubcoreMesh(core_axis_name="c", subcore_axis_name="s")


def _make_deg_kernel():
    @functools.partial(
        pl.kernel,
        out_type=jax.ShapeDtypeStruct((2, NP), F32),
        mesh=_mesh(),
        scratch_types=[
            pltpu.VMEM_SHARED((NP,), F32),
            pltpu.VMEM((NB, BLK), jnp.int32),
            pltpu.VMEM((SLICE,), F32),
            pltpu.VMEM((BLK,), F32),
            pltpu.SemaphoreType.DMA,
        ],
        compiler_params=pltpu.CompilerParams(use_tc_tiling_on_sc=False),
    )
    def deg_kernel(eib, out, acc, dst_v, zero_v, ones_v, sem):
        cid = lax.axis_index("c")
        sid = lax.axis_index("s")
        wid = cid * 16 + sid
        r0 = sid * SLICE

        def zloop(i, carry):
            zero_v[pl.ds(i * 16, 16)] = jnp.zeros((16,), F32)
            return carry

        lax.fori_loop(0, SLICE // 16, zloop, 0)

        def oloop(i, carry):
            ones_v[pl.ds(i * 16, 16)] = jnp.ones((16,), F32)
            return carry

        lax.fori_loop(0, BLK // 16, oloop, 0)

        pltpu.sync_copy(zero_v, acc.at[pl.ds(r0, SLICE)])
        pltpu.sync_copy(eib.at[1, wid], dst_v)
        plsc.subcore_barrier()

        def fire(j, carry):
            pltpu.async_copy(ones_v, acc.at[dst_v.at[j]], sem, add=True)
            return carry

        lax.fori_loop(0, NB, fire, 0)

        def drain(j, carry):
            pltpu.make_async_copy(ones_v, acc.at[dst_v.at[j]], sem).wait()
            return carry

        lax.fori_loop(0, NB, drain, 0)
        plsc.subcore_barrier()
        pltpu.sync_copy(acc.at[pl.ds(r0, SLICE)], out.at[cid, pl.ds(r0, SLICE)])

    return deg_kernel


def _make_scatter_kernel(D, K, blk, nb):
    H = nb // K
    NG = 8
    G = 4
    L = 4
    assert H % NG == 0 and H * K == nb and G + L <= NG

    @functools.partial(
        pl.kernel,
        out_type=jax.ShapeDtypeStruct((2, NP, D), F32),
        mesh=_mesh(),
        scratch_types=[
            pltpu.VMEM_SHARED((NP, D), F32),
            pltpu.VMEM((nb, blk), jnp.int32),
            pltpu.VMEM((nb, blk), jnp.int32),
            pltpu.VMEM((NG, K, blk, D), F32),
            [pltpu.SemaphoreType.DMA] * NG,
            [pltpu.SemaphoreType.DMA] * NG,
        ],
        compiler_params=pltpu.CompilerParams(use_tc_tiling_on_sc=False),
    )
    def scatter_kernel(g_hbm, eib, out, acc, src_v, dst_v, rows_v,
                       gsems, ssems):
        cid = lax.axis_index("c")
        sid = lax.axis_index("s")
        wid = cid * 16 + sid
        r0 = sid * SLICE

        def zrow(r, carry):
            def zcol(cc, carry2):
                rows_v[0, 0, r, pl.ds(cc * 16, 16)] = jnp.zeros((16,), F32)
                return carry2

            lax.fori_loop(0, D // 16, zcol, 0)
            return carry

        lax.fori_loop(0, blk, zrow, 0)

        for k in range(SLICE // blk):
            pltpu.sync_copy(rows_v.at[0, 0], acc.at[pl.ds(r0 + k * blk, blk)])

        pltpu.sync_copy(eib.at[0, wid], src_v)
        pltpu.sync_copy(eib.at[1, wid], dst_v)
        plsc.subcore_barrier()

        def fire_gathers(c, grp):
            for k in range(K):
                pltpu.async_copy(g_hbm.at[src_v.at[c * K + k]],
                                 rows_v.at[grp, k], gsems[grp])

        def wait_gathers(c, grp):
            for k in range(K):
                pltpu.make_async_copy(g_hbm.at[src_v.at[c * K + k]],
                                      rows_v.at[grp, k], gsems[grp]).wait()

        def fire_scatters(c, grp):
            for k in range(K):
                pltpu.async_copy(rows_v.at[grp, k],
                                 acc.at[dst_v.at[c * K + k]], ssems[grp],
                                 add=True)

        def drain_scatters(c, grp):
            for k in range(K):
                pltpu.make_async_copy(rows_v.at[grp, k],
                                      acc.at[dst_v.at[c * K + k]],
                                      ssems[grp]).wait()

        for c in range(G):
            fire_gathers(c, c % NG)

        def phase_block(t, carry):
            base = NG * t
            for q in range(NG):
                p = base + q
                wait_gathers(p, q)
                fire_scatters(p, q)

                @pl.when(p >= L)
                def _():
                    drain_scatters(p - L, (q - L) % NG)

                @pl.when(p + G < H)
                def _():
                    fire_gathers(p + G, (q + G) % NG)

            return carry

        lax.fori_loop(0, H // NG, phase_block, 0)
        for p in range(H - L, H):
            drain_scatters(p, p % NG)
        plsc.subcore_barrier()
        pltpu.sync_copy(acc.at[pl.ds(r0, SLICE)], out.at[cid, pl.ds(r0, SLICE)])

    return scatter_kernel


_deg = _make_deg_kernel()
_scat16 = _make_scatter_kernel(16, 2, BLK, NB)
_scat64 = _make_scatter_kernel(64, 1, BLK, NB)

_R = 2560
_G = NP // _R


def _tc1(x, W1, degp):
    R = 2560
    G = NP // R

    def body(x_ref, w_ref, dg_ref, o_ref):
        d = dg_ref[0, :] + dg_ref[1, :] + 1.0
        dinv = lax.rsqrt(d)
        h = jnp.dot(x_ref[...], w_ref[...], preferred_element_type=F32)
        o_ref[...] = h * dinv[:, None]

    return pl.pallas_call(
        body,
        grid=(G,),
        in_specs=[
            pl.BlockSpec((R, 128), lambda i: (i, 0)),
            pl.BlockSpec((128, 16), lambda i: (0, 0)),
            pl.BlockSpec((2, R), lambda i: (0, i)),
        ],
        out_specs=pl.BlockSpec((R, 16), lambda i: (i, 0)),
        out_shape=jax.ShapeDtypeStruct((NP, 16), F32),
    )(x, W1, degp)


def _tc2(s1p, g1, degp, W2, b1):
    def body(s_ref, g_ref, dg_ref, w_ref, b_ref, o_ref):
        d = dg_ref[0, :] + dg_ref[1, :] + 1.0
        dinv = lax.rsqrt(d)
        z = (s_ref[0] + s_ref[1] + g_ref[...]) * dinv[:, None] + b_ref[...]
        h = jnp.maximum(z, 0.0)
        h2 = jnp.dot(h, w_ref[...], preferred_element_type=F32)
        o_ref[...] = h2 * dinv[:, None]

    return pl.pallas_call(
        body,
        grid=(_G,),
        in_specs=[
            pl.BlockSpec((2, _R, 16), lambda i: (0, i, 0)),
            pl.BlockSpec((_R, 16), lambda i: (i, 0)),
            pl.BlockSpec((2, _R), lambda i: (0, i)),
            pl.BlockSpec((16, 64), lambda i: (0, 0)),
            pl.BlockSpec((1, 16), lambda i: (0, 0)),
        ],
        out_specs=pl.BlockSpec((_R, 64), lambda i: (i, 0)),
        out_shape=jax.ShapeDtypeStruct((NP, 64), F32),
    )(s1p, g1, degp, W2, b1)


def _tc3(s2p, g2, degp, b2):
    def body(s_ref, g_ref, dg_ref, b_ref, o_ref):
        d = dg_ref[0, :] + dg_ref[1, :] + 1.0
        dinv = lax.rsqrt(d)
        z = (s_ref[0] + s_ref[1] + g_ref[...]) * dinv[:, None] + b_ref[...]
        m = jnp.max(z, axis=1, keepdims=True)
        e = jnp.exp(z - m)
        lse = jnp.log(jnp.sum(e, axis=1, keepdims=True))
        o_ref[...] = z - m - lse

    return pl.pallas_call(
        body,
        grid=(_G,),
        in_specs=[
            pl.BlockSpec((2, _R, 64), lambda i: (0, i, 0)),
            pl.BlockSpec((_R, 64), lambda i: (i, 0)),
            pl.BlockSpec((2, _R), lambda i: (0, i)),
            pl.BlockSpec((1, 64), lambda i: (0, 0)),
        ],
        out_specs=pl.BlockSpec((_R, 64), lambda i: (i, 0)),
        out_shape=jax.ShapeDtypeStruct((N, 64), F32),
    )(s2p, g2, degp, b2)


def kernel(x, edge_index, W1, b1, W2, b2):
    ei = edge_index.astype(jnp.int32)
    npad = NW * EWP - E
    pad = jnp.arange(npad, dtype=jnp.int32)
    eip = jnp.concatenate(
        [ei, jnp.stack([pad % N, N + pad % (NP - N)])], axis=1)
    eib = eip.reshape(2, NW, NB, BLK)

    degp = _deg(eib)
    g1 = _tc1(x, W1, degp)
    s1p = _scat16(g1, eib)
    g2 = _tc2(s1p, g1, degp, W2, b1.reshape(1, 16))
    s2p = _scat64(g2, eib)
    return _tc3(s2p, g2, degp, b2.reshape(1, 64))


# speedup vs baseline: 1.0021x; 1.0021x over previous
import functools

import jax
import jax.numpy as jnp
from jax import lax
from jax.experimental import pallas as pl
from jax.experimental.pallas import tpu as pltpu
from jax.experimental.pallas import tpu_sc as plsc

N = 10000
NP = 10240
E = 320000
NW = 32
EW = E // NW
BLK = 128
NB = 80
EWP = NB * BLK
SLICE = NP // 16
F32 = jnp.float32


def _mesh():
    return plsc.VectorSubcoreMesh(core_axis_name="c", subcore_axis_name="s")


def _make_deg_kernel():
    @functools.partial(
        pl.kernel,
        out_type=jax.ShapeDtypeStruct((2, NP), F32),
        mesh=_mesh(),
        scratch_types=[
            pltpu.VMEM_SHARED((NP,), F32),
            pltpu.VMEM((NB, BLK), jnp.int32),
            pltpu.VMEM((SLICE,), F32),
            pltpu.VMEM((BLK,), F32),
            pltpu.SemaphoreType.DMA,
        ],
        compiler_params=pltpu.CompilerParams(use_tc_tiling_on_sc=False),
    )
    def deg_kernel(eib, out, acc, dst_v, zero_v, ones_v, sem):
        cid = lax.axis_index("c")
        sid = lax.axis_index("s")
        wid = cid * 16 + sid
        r0 = sid * SLICE

        def zloop(i, carry):
            zero_v[pl.ds(i * 16, 16)] = jnp.zeros((16,), F32)
            return carry

        lax.fori_loop(0, SLICE // 16, zloop, 0)

        def oloop(i, carry):
            ones_v[pl.ds(i * 16, 16)] = jnp.ones((16,), F32)
            return carry

        lax.fori_loop(0, BLK // 16, oloop, 0)

        pltpu.sync_copy(zero_v, acc.at[pl.ds(r0, SLICE)])
        pltpu.sync_copy(eib.at[1, wid], dst_v)
        plsc.subcore_barrier()

        def fire(j, carry):
            pltpu.async_copy(ones_v, acc.at[dst_v.at[j]], sem, add=True)
            return carry

        lax.fori_loop(0, NB, fire, 0)

        def drain(j, carry):
            pltpu.make_async_copy(ones_v, acc.at[dst_v.at[j]], sem).wait()
            return carry

        lax.fori_loop(0, NB, drain, 0)
        plsc.subcore_barrier()
        pltpu.sync_copy(acc.at[pl.ds(r0, SLICE)], out.at[cid, pl.ds(r0, SLICE)])

    return deg_kernel


def _make_scatter_kernel(D, K, blk, nb):
    H = nb // K
    NG = 8
    G = 4
    L = 4
    assert H % NG == 0 and H * K == nb and G + L <= NG

    @functools.partial(
        pl.kernel,
        out_type=jax.ShapeDtypeStruct((2, NP, D), F32),
        mesh=_mesh(),
        scratch_types=[
            pltpu.VMEM_SHARED((NP, D), F32),
            pltpu.VMEM((nb, blk), jnp.int32),
            pltpu.VMEM((nb, blk), jnp.int32),
            pltpu.VMEM((NG, K, blk, D), F32),
            [pltpu.SemaphoreType.DMA] * NG,
            [pltpu.SemaphoreType.DMA] * NG,
        ],
        compiler_params=pltpu.CompilerParams(use_tc_tiling_on_sc=False),
    )
    def scatter_kernel(g_hbm, eib, out, acc, src_v, dst_v, rows_v,
                       gsems, ssems):
        cid = lax.axis_index("c")
        sid = lax.axis_index("s")
        wid = cid * 16 + sid
        r0 = sid * SLICE

        def zrow(r, carry):
            def zcol(cc, carry2):
                rows_v[0, 0, r, pl.ds(cc * 16, 16)] = jnp.zeros((16,), F32)
                return carry2

            lax.fori_loop(0, D // 16, zcol, 0)
            return carry

        lax.fori_loop(0, blk, zrow, 0)

        for k in range(SLICE // blk):
            pltpu.sync_copy(rows_v.at[0, 0], acc.at[pl.ds(r0 + k * blk, blk)])

        pltpu.sync_copy(eib.at[0, wid], src_v)
        pltpu.sync_copy(eib.at[1, wid], dst_v)
        plsc.subcore_barrier()

        def fire_gathers(c, grp):
            for k in range(K):
                pltpu.async_copy(g_hbm.at[src_v.at[c * K + k]],
                                 rows_v.at[grp, k], gsems[grp])

        def wait_gathers(c, grp):
            for k in range(K):
                pltpu.make_async_copy(g_hbm.at[src_v.at[c * K + k]],
                                      rows_v.at[grp, k], gsems[grp]).wait()

        def fire_scatters(c, grp):
            for k in range(K):
                pltpu.async_copy(rows_v.at[grp, k],
                                 acc.at[dst_v.at[c * K + k]], ssems[grp],
                                 add=True)

        def drain_scatters(c, grp):
            for k in range(K):
                pltpu.make_async_copy(rows_v.at[grp, k],
                                      acc.at[dst_v.at[c * K + k]],
                                      ssems[grp]).wait()

        for c in range(G):
            fire_gathers(c, c % NG)

        def phase_block(t, carry):
            base = NG * t
            for q in range(NG):
                p = base + q
                wait_gathers(p, q)
                fire_scatters(p, q)

                @pl.when(p >= L)
                def _():
                    drain_scatters(p - L, (q - L) % NG)

                @pl.when(p + G < H)
                def _():
                    fire_gathers(p + G, (q + G) % NG)

            return carry

        lax.fori_loop(0, H // NG, phase_block, 0)
        for p in range(H - L, H):
            drain_scatters(p, p % NG)
        plsc.subcore_barrier()
        pltpu.sync_copy(acc.at[pl.ds(r0, SLICE)], out.at[cid, pl.ds(r0, SLICE)])

    return scatter_kernel


_deg = _make_deg_kernel()
_scat16 = _make_scatter_kernel(16, 2, BLK, NB)
_scat64 = _make_scatter_kernel(64, 1, BLK, NB)

_R = 2560
_G = NP // _R


def _tc1(x, W1, degp):
    R = 2560
    G = NP // R

    def body(x_ref, w_ref, dg_ref, o_ref):
        d = dg_ref[0, :] + dg_ref[1, :] + 1.0
        dinv = lax.rsqrt(d)
        h = jnp.dot(x_ref[...], w_ref[...], preferred_element_type=F32)
        o_ref[...] = h * dinv[:, None]

    return pl.pallas_call(
        body,
        grid=(G,),
        in_specs=[
            pl.BlockSpec((R, 128), lambda i: (i, 0)),
            pl.BlockSpec((128, 16), lambda i: (0, 0)),
            pl.BlockSpec((2, R), lambda i: (0, i)),
        ],
        out_specs=pl.BlockSpec((R, 16), lambda i: (i, 0)),
        out_shape=jax.ShapeDtypeStruct((NP, 16), F32),
    )(x, W1, degp)


def _tc2(s1p, g1, degp, W2, b1):
    def body(s_ref, g_ref, dg_ref, w_ref, b_ref, o_ref):
        d = dg_ref[0, :] + dg_ref[1, :] + 1.0
        dinv = lax.rsqrt(d)
        z = (s_ref[0] + s_ref[1] + g_ref[...]) * dinv[:, None] + b_ref[...]
        h = jnp.maximum(z, 0.0)
        h2 = jnp.dot(h, w_ref[...], preferred_element_type=F32)
        o_ref[...] = h2 * dinv[:, None]

    return pl.pallas_call(
        body,
        grid=(_G,),
        in_specs=[
            pl.BlockSpec((2, _R, 16), lambda i: (0, i, 0)),
            pl.BlockSpec((_R, 16), lambda i: (i, 0)),
            pl.BlockSpec((2, _R), lambda i: (0, i)),
            pl.BlockSpec((16, 64), lambda i: (0, 0)),
            pl.BlockSpec((1, 16), lambda i: (0, 0)),
        ],
        out_specs=pl.BlockSpec((_R, 64), lambda i: (i, 0)),
        out_shape=jax.ShapeDtypeStruct((NP, 64), F32),
    )(s1p, g1, degp, W2, b1)


def _tc3(s2p, g2, degp, b2):
    def body(s_ref, g_ref, dg_ref, b_ref, o_ref):
        d = dg_ref[0, :] + dg_ref[1, :] + 1.0
        dinv = lax.rsqrt(d)
        z = (s_ref[0] + s_ref[1] + g_ref[...]) * dinv[:, None] + b_ref[...]
        m = jnp.max(z, axis=1, keepdims=True)
        e = jnp.exp(z - m)
        lse = jnp.log(jnp.sum(e, axis=1, keepdims=True))
        o_ref[...] = z - m - lse

    return pl.pallas_call(
        body,
        grid=(_G,),
        in_specs=[
            pl.BlockSpec((2, _R, 64), lambda i: (0, i, 0)),
            pl.BlockSpec((_R, 64), lambda i: (i, 0)),
            pl.BlockSpec((2, _R), lambda i: (0, i)),
            pl.BlockSpec((1, 64), lambda i: (0, 0)),
        ],
        out_specs=pl.BlockSpec((_R, 64), lambda i: (i, 0)),
        out_shape=jax.ShapeDtypeStruct((N, 64), F32),
    )(s2p, g2, degp, b2)


def kernel(x, edge_index, W1, b1, W2, b2):
    ei = edge_index.astype(jnp.int32)
    npad = NW * EWP - E
    pad = jnp.arange(npad, dtype=jnp.int32)
    eip = jnp.concatenate(
        [ei, jnp.stack([pad % N, N + pad % (NP - N)])], axis=1)
    eib = eip.reshape(2, NW, NB, BLK)

    degp = _deg(eib)
    g1 = _tc1(x, W1, degp)
    s1p = _scat16(g1, eib)
    g2 = _tc2(s1p, g1, degp, W2, b1.reshape(1, 16))
    s2p = _scat64(g2, eib)
    return _tc3(s2p, g2, degp, b2.reshape(1, 64))
